# baseline (speedup 1.0000x reference)
import functools

import jax
import jax.numpy as jnp
from jax import lax
from jax.experimental import pallas as pl
from jax.experimental.pallas import tpu as pltpu
from jax.experimental.pallas import tpu_sc as plsc

N = 10000
E = 320000
DF = 128
C = 128
DE = 16
EO = 16
ATT = 2 * C + EO
NEG = 0.2

NC = 2
NS = 16
NW = NC * NS
EPT = E // NW
SUB = 80
NSUB = 5
CH = SUB * NSUB
NCH = EPT // CH
RPT = 624
RPT_LAST = N - 15 * RPT
BCH = 48
NBCH = RPT // BCH
SPT = N // 10

EB = 8000
_PREC = lax.Precision.HIGHEST


def _leaky(v):
    return jnp.where(v >= 0, v, NEG * v)


def _node_body(x_ref, wl_ref, bl_ref, att_ref, xl_ref, si_ref, sj_ref):
    xl = lax.dot_general(x_ref[...], wl_ref[...], (((1,), (1,)), ((), ())),
                         precision=_PREC) + bl_ref[...][None, :]
    xl_ref[...] = xl
    lr = _leaky(xl)
    a = att_ref[...].reshape(ATT)
    si_ref[...] = lr @ a[:C]
    sj_ref[...] = lr @ a[C:2 * C]


def _se_body(ea_ref, wbd_ref, bt_ref, att_ref, sel_ref, se_ref):
    ea = ea_ref[...].reshape(EBS, 8 * DE)
    e = lax.dot_general(ea, wbd_ref[...], (((1,), (0,)), ((), ())),
                        precision=_PREC) + bt_ref[...][None, :]
    se_ref[...] = lax.dot_general(_leaky(e) * att_ref[...][None, :],
                                  sel_ref[...], (((1,), (0,)), ((), ())),
                                  precision=_PREC)


def _eo_body(ea_ref, wbd_ref, bt_ref, wbdo_ref, bot_ref, eo_ref):
    ea = ea_ref[...].reshape(EBO, 8 * DE)
    e = lax.dot_general(ea, wbd_ref[...], (((1,), (0,)), ((), ())),
                        precision=_PREC) + bt_ref[...][None, :]
    eo = lax.dot_general(e, wbdo_ref[...], (((1,), (0,)), ((), ())),
                         precision=_PREC) + bot_ref[...][None, :]
    eo_ref[...] = eo.reshape(EBO, 8, EO)


def _out_body(acc_ref, s0_ref, s1_ref, wno_ref, bno_ref, out_ref):
    acc = acc_ref[0] + acc_ref[1]
    ssum = s0_ref[...] + s1_ref[...]
    scale = 1.0 / (ssum + 1e-16)
    out_ref[...] = lax.dot_general(acc * scale[:, None], wno_ref[...],
                                   (((1,), (1,)), ((), ())),
                                   precision=_PREC) + bno_ref[...][None, :]


_node_call = pl.pallas_call(
    _node_body,
    out_shape=[jax.ShapeDtypeStruct((N, C), jnp.float32),
               jax.ShapeDtypeStruct((N,), jnp.float32),
               jax.ShapeDtypeStruct((N,), jnp.float32)],
)

EP = E // 8
EBS = 1000

_se_call = pl.pallas_call(
    _se_body,
    grid=(EP // EBS,),
    in_specs=[pl.BlockSpec((EBS, 8, DE), lambda i: (i, 0, 0)),
              pl.BlockSpec((128, 128), lambda i: (0, 0)),
              pl.BlockSpec((128,), lambda i: (0,)),
              pl.BlockSpec((128,), lambda i: (0,)),
              pl.BlockSpec((128, 8), lambda i: (0, 0))],
    out_specs=pl.BlockSpec((EBS, 8), lambda i: (i, 0)),
    out_shape=jax.ShapeDtypeStruct((EP, 8), jnp.float32),
)

EBO = 1000

_eo_call = pl.pallas_call(
    _eo_body,
    grid=(EP // EBO,),
    in_specs=[pl.BlockSpec((EBO, 8, DE), lambda i: (i, 0, 0)),
              pl.BlockSpec((128, 128), lambda i: (0, 0)),
              pl.BlockSpec((128,), lambda i: (0,)),
              pl.BlockSpec((128, 128), lambda i: (0, 0)),
              pl.BlockSpec((128,), lambda i: (0,))],
    out_specs=pl.BlockSpec((EBO, 8, EO), lambda i: (i, 0, 0)),
    out_shape=jax.ShapeDtypeStruct((EP, 8, EO), jnp.float32),
)

_out_call = pl.pallas_call(
    _out_body,
    out_shape=jax.ShapeDtypeStruct((N, C), jnp.float32),
)


def _scale_sub(rows_ref, p2_ref, j):
    def scale_grp(g, c2):
        p16 = p2_ref[j, pl.ds(g * 16, 16)]
        for l in range(16):
            b = g * 16 + l
            pb = p16[l]
            for cc in range(C // 16):
                rows_ref[b, pl.ds(cc * 16, 16)] = (
                    rows_ref[b, pl.ds(cc * 16, 16)] * pb)
        return c2

    lax.fori_loop(0, SUB // 16, scale_grp, 0)


def _sc_body(xl_hbm, si_hbm, sj_hbm, se_hbm, src_hbm, dst3_hbm, zr_hbm,
             zs_hbm, acc_hbm, s0_hbm, s1_hbm,
             si_v, sj_v, src_v, dst2_v, se_v, p2_v, rows_a, rows_b, ssb_v,
             acc_sh, ssum_sh, gsem, sem_a, sem_b, psem):
    cid = lax.axis_index("c")
    sid = lax.axis_index("s")
    wid = cid * NS + sid

    pltpu.sync_copy(si_hbm, si_v)
    pltpu.sync_copy(sj_hbm, sj_v)

    pltpu.sync_copy(zr_hbm, rows_a.at[pl.ds(0, BCH)])

    def zero_blk(j, carry):
        pltpu.sync_copy(rows_a.at[pl.ds(0, BCH)],
                        acc_sh.at[pl.ds(sid * RPT + j * BCH, BCH)])
        return carry

    lax.fori_loop(0, NBCH, zero_blk, 0)

    @pl.when(sid == 15)
    def _():
        pltpu.sync_copy(rows_a.at[pl.ds(0, 16)],
                        acc_sh.at[pl.ds(15 * RPT + NBCH * BCH, 16)])

    @pl.when(sid < 10)
    def _():
        pltpu.sync_copy(zs_hbm, ssb_v)
        pltpu.sync_copy(ssb_v, ssum_sh.at[pl.ds(sid * SPT, SPT)])

    plsc.subcore_barrier()

    bufs = (rows_a, rows_b)
    sems = (sem_a, sem_b)

    def chunk(ch, carry):
        base = wid * EPT + ch * CH
        l0 = pltpu.async_copy(src_hbm.at[pl.ds(base, CH)], src_v, gsem)
        l1 = pltpu.async_copy(dst3_hbm.at[wid * NCH + ch], dst2_v, gsem)
        l2 = pltpu.async_copy(se_hbm.at[pl.ds(base, CH)], se_v, gsem)
        l0.wait()
        l1.wait()
        l2.wait()
        g0 = pltpu.async_copy(xl_hbm.at[src_v.at[pl.ds(0, SUB)]],
                              rows_a, gsem)
        g1 = pltpu.async_copy(xl_hbm.at[src_v.at[pl.ds(SUB, SUB)]],
                              rows_b, gsem)
        for g in range(CH // 16):
            s = g * 16
            isrc = src_v[pl.ds(s, 16)]
            idst = dst2_v[g // (SUB // 16), pl.ds((s % SUB), 16)]
            sj = plsc.load_gather(sj_v, [isrc])
            si = plsc.load_gather(si_v, [idst])
            p2_v[g // (SUB // 16), pl.ds(s % SUB, 16)] = (
                jnp.exp(si + sj + se_v[pl.ds(s, 16)]))
        ph = [pltpu.async_copy(p2_v.at[j], ssum_sh.at[dst2_v.at[j]], psem,
                               add=True)
              for j in range(NSUB)]
        gh = [g0, g1, None, None, None]
        sh = [None] * NSUB
        for j in range(NSUB):
            b = j % 2
            if j >= 2:
                sh[j - 2].wait()
                gh[j] = pltpu.async_copy(
                    xl_hbm.at[src_v.at[pl.ds(j * SUB, SUB)]], bufs[b],
                    gsem)
            gh[j].wait()
            _scale_sub(bufs[b], p2_v, j)
            sh[j] = pltpu.async_copy(bufs[b], acc_sh.at[dst2_v.at[j]],
                                     sems[b], add=True)
        sh[NSUB - 2].wait()
        sh[NSUB - 1].wait()
        for h in ph:
            h.wait()
        return carry

    lax.fori_loop(0, NCH, chunk, 0)

    plsc.subcore_barrier()

    def drain_blk(j, carry):
        off = sid * RPT + j * BCH
        pltpu.sync_copy(acc_sh.at[pl.ds(off, BCH)], rows_a.at[pl.ds(0, BCH)])
        pltpu.sync_copy(rows_a.at[pl.ds(0, BCH)],
                        acc_hbm.at[cid, pl.ds(off, BCH)])
        return carry

    lax.fori_loop(0, NBCH, drain_blk, 0)

    @pl.when(sid == 15)
    def _():
        off = 15 * RPT + NBCH * BCH
        pltpu.sync_copy(acc_sh.at[pl.ds(off, 16)], rows_a.at[pl.ds(0, 16)])
        pltpu.sync_copy(rows_a.at[pl.ds(0, 16)],
                        acc_hbm.at[cid, pl.ds(off, 16)])

    @pl.when(sid < 10)
    def _():
        pltpu.sync_copy(ssum_sh.at[pl.ds(sid * SPT, SPT)], ssb_v)

    @pl.when(jnp.logical_and(sid < 10, cid == 0))
    def _():
        pltpu.sync_copy(ssb_v, s0_hbm.at[pl.ds(sid * SPT, SPT)])

    @pl.when(jnp.logical_and(sid < 10, cid == 1))
    def _():
        pltpu.sync_copy(ssb_v, s1_hbm.at[pl.ds(sid * SPT, SPT)])


@functools.cache
def _sc_call():
    return pl.kernel(
        _sc_body,
        mesh=plsc.VectorSubcoreMesh(core_axis_name="c", subcore_axis_name="s"),
        compiler_params=pltpu.CompilerParams(needs_layout_passes=False),
        out_type=[jax.ShapeDtypeStruct((NC, N, C), jnp.float32),
                  jax.ShapeDtypeStruct((N,), jnp.float32),
                  jax.ShapeDtypeStruct((N,), jnp.float32)],
        scratch_types=[
            pltpu.VMEM((N,), jnp.float32),
            pltpu.VMEM((N,), jnp.float32),
            pltpu.VMEM((CH,), jnp.int32),
            pltpu.VMEM((NSUB, SUB), jnp.int32),
            pltpu.VMEM((CH,), jnp.float32),
            pltpu.VMEM((NSUB, SUB), jnp.float32),
            pltpu.VMEM((SUB, C), jnp.float32),
            pltpu.VMEM((SUB, C), jnp.float32),
            pltpu.VMEM((SPT,), jnp.float32),
            pltpu.VMEM_SHARED((N, C), jnp.float32),
            pltpu.VMEM_SHARED((N,), jnp.float32),
            pltpu.SemaphoreType.DMA,
            pltpu.SemaphoreType.DMA,
            pltpu.SemaphoreType.DMA,
            pltpu.SemaphoreType.DMA,
        ],
    )


def kernel(x, edge_attr, edge_index, W_l, b_l, W_e, b_e, att, W_no, b_no,
           W_eo, b_eo):
    ea3 = edge_attr.reshape(EP, 8, DE)
    eye8 = jnp.eye(8, dtype=jnp.float32)
    wbd = jnp.kron(eye8, W_e.T)
    bt = jnp.tile(b_e, 8)
    att_flat = att.reshape(ATT)
    att_t = jnp.tile(att_flat[2 * C:], 8)
    sel = jnp.kron(eye8, jnp.ones((DE, 1), jnp.float32))
    wbdo = jnp.kron(eye8, W_eo.T)
    bot = jnp.tile(b_eo, 8)

    xl, si, sj = _node_call(x, W_l, b_l, att)
    se8 = _se_call(ea3, wbd, bt, att_t, sel)
    se = se8.reshape(E)
    src = edge_index[0]
    dst3 = edge_index[1].reshape(E // CH, NSUB, SUB)
    zr = jnp.zeros((BCH, C), jnp.float32)
    zs = jnp.zeros((SPT,), jnp.float32)
    acc2, ssum0, ssum1 = _sc_call()(xl, si, sj, se, src, dst3, zr, zs)
    eo3 = _eo_call(ea3, wbd, bt, wbdo, bot)
    e_out = eo3.reshape(E, EO)
    out = _out_call(acc2, ssum0, ssum1, W_no, b_no)
    return (out, e_out)


# speedup vs baseline: 1.1557x; 1.1557x over previous
import functools

import jax
import jax.numpy as jnp
from jax import lax
from jax.experimental import pallas as pl
from jax.experimental.pallas import tpu as pltpu
from jax.experimental.pallas import tpu_sc as plsc

N = 10000
E = 320000
DF = 128
C = 128
DE = 16
EO = 16
ATT = 2 * C + EO
NEG = 0.2

NC = 2
NS = 16
NW = NC * NS
EPT = E // NW
SUB = 80
NSUB = 5
CH = SUB * NSUB
NCH = EPT // CH
RPT = 624
RPT_LAST = N - 15 * RPT
BCH = 48
NBCH = RPT // BCH
SPT = N // 10

EB = 8000
_PREC = lax.Precision.HIGHEST
_EPREC = lax.Precision.DEFAULT


def _leaky(v):
    return jnp.where(v >= 0, v, NEG * v)


def _node_body(x_ref, wl_ref, bl_ref, att_ref, xl_ref, si_ref, sj_ref):
    xl = lax.dot_general(x_ref[...], wl_ref[...], (((1,), (1,)), ((), ())),
                         precision=_PREC) + bl_ref[...][None, :]
    xl_ref[...] = xl
    lr = _leaky(xl)
    a = att_ref[...].reshape(ATT)
    si_ref[...] = lr @ a[:C]
    sj_ref[...] = lr @ a[C:2 * C]


def _se_body(ea_ref, wbd_ref, bt_ref, att_ref, sel_ref, se_ref):
    ea = ea_ref[...].reshape(EBS, 8 * DE)
    e = lax.dot_general(ea, wbd_ref[...], (((1,), (0,)), ((), ())),
                        precision=_EPREC) + bt_ref[...][None, :]
    se_ref[...] = lax.dot_general(_leaky(e) * att_ref[...][None, :],
                                  sel_ref[...], (((1,), (0,)), ((), ())),
                                  precision=_EPREC)


def _eo_body(ea_ref, wbd_ref, bt_ref, wbdo_ref, bot_ref, eo_ref):
    ea = ea_ref[...].reshape(EBO, 8 * DE)
    e = lax.dot_general(ea, wbd_ref[...], (((1,), (0,)), ((), ())),
                        precision=_EPREC) + bt_ref[...][None, :]
    eo = lax.dot_general(e, wbdo_ref[...], (((1,), (0,)), ((), ())),
                         precision=_EPREC) + bot_ref[...][None, :]
    eo_ref[...] = eo.reshape(EBO, 8, EO)


def _out_body(acc_ref, s0_ref, s1_ref, wno_ref, bno_ref, out_ref):
    acc = acc_ref[0] + acc_ref[1]
    ssum = s0_ref[...] + s1_ref[...]
    scale = 1.0 / (ssum + 1e-16)
    out_ref[...] = lax.dot_general(acc * scale[:, None], wno_ref[...],
                                   (((1,), (1,)), ((), ())),
                                   precision=_PREC) + bno_ref[...][None, :]


_node_call = pl.pallas_call(
    _node_body,
    out_shape=[jax.ShapeDtypeStruct((N, C), jnp.float32),
               jax.ShapeDtypeStruct((N,), jnp.float32),
               jax.ShapeDtypeStruct((N,), jnp.float32)],
)

EP = E // 8
EBS = 1000

_se_call = pl.pallas_call(
    _se_body,
    grid=(EP // EBS,),
    in_specs=[pl.BlockSpec((EBS, 8, DE), lambda i: (i, 0, 0)),
              pl.BlockSpec((128, 128), lambda i: (0, 0)),
              pl.BlockSpec((128,), lambda i: (0,)),
              pl.BlockSpec((128,), lambda i: (0,)),
              pl.BlockSpec((128, 8), lambda i: (0, 0))],
    out_specs=pl.BlockSpec((EBS, 8), lambda i: (i, 0)),
    out_shape=jax.ShapeDtypeStruct((EP, 8), jnp.float32),
)

EBO = 1000

_eo_call = pl.pallas_call(
    _eo_body,
    grid=(EP // EBO,),
    in_specs=[pl.BlockSpec((EBO, 8, DE), lambda i: (i, 0, 0)),
              pl.BlockSpec((128, 128), lambda i: (0, 0)),
              pl.BlockSpec((128,), lambda i: (0,)),
              pl.BlockSpec((128, 128), lambda i: (0, 0)),
              pl.BlockSpec((128,), lambda i: (0,))],
    out_specs=pl.BlockSpec((EBO, 8, EO), lambda i: (i, 0, 0)),
    out_shape=jax.ShapeDtypeStruct((EP, 8, EO), jnp.float32),
)

_out_call = pl.pallas_call(
    _out_body,
    out_shape=jax.ShapeDtypeStruct((N, C), jnp.float32),
)


def _scale_sub(rows_ref, p2_ref, j):
    def scale_grp(g, c2):
        p16 = p2_ref[j, pl.ds(g * 16, 16)]
        for l in range(16):
            b = g * 16 + l
            pb = p16[l]
            for cc in range(C // 16):
                rows_ref[b, pl.ds(cc * 16, 16)] = (
                    rows_ref[b, pl.ds(cc * 16, 16)] * pb)
        return c2

    lax.fori_loop(0, SUB // 16, scale_grp, 0)


def _sc_body(xl_hbm, si_hbm, sj_hbm, se_hbm, src_hbm, dst3_hbm, zr_hbm,
             zs_hbm, acc_hbm, s0_hbm, s1_hbm,
             si_v, sj_v, src_v, dst2_v, se_v, p2_v, rows_a, rows_b, ssb_v,
             acc_sh, ssum_sh, gsem, sem_a, sem_b, psem):
    cid = lax.axis_index("c")
    sid = lax.axis_index("s")
    wid = cid * NS + sid

    pltpu.sync_copy(si_hbm, si_v)
    pltpu.sync_copy(sj_hbm, sj_v)

    pltpu.sync_copy(zr_hbm, rows_a.at[pl.ds(0, BCH)])

    def zero_blk(j, carry):
        pltpu.sync_copy(rows_a.at[pl.ds(0, BCH)],
                        acc_sh.at[pl.ds(sid * RPT + j * BCH, BCH)])
        return carry

    lax.fori_loop(0, NBCH, zero_blk, 0)

    @pl.when(sid == 15)
    def _():
        pltpu.sync_copy(rows_a.at[pl.ds(0, 16)],
                        acc_sh.at[pl.ds(15 * RPT + NBCH * BCH, 16)])

    @pl.when(sid < 10)
    def _():
        pltpu.sync_copy(zs_hbm, ssb_v)
        pltpu.sync_copy(ssb_v, ssum_sh.at[pl.ds(sid * SPT, SPT)])

    plsc.subcore_barrier()

    bufs = (rows_a, rows_b)
    sems = (sem_a, sem_b)

    def chunk(ch, carry):
        base = wid * EPT + ch * CH
        l0 = pltpu.async_copy(src_hbm.at[pl.ds(base, CH)], src_v, gsem)
        l1 = pltpu.async_copy(dst3_hbm.at[wid * NCH + ch], dst2_v, gsem)
        l2 = pltpu.async_copy(se_hbm.at[pl.ds(base, CH)], se_v, gsem)
        l0.wait()
        l1.wait()
        l2.wait()
        g0 = pltpu.async_copy(xl_hbm.at[src_v.at[pl.ds(0, SUB)]],
                              rows_a, gsem)
        g1 = pltpu.async_copy(xl_hbm.at[src_v.at[pl.ds(SUB, SUB)]],
                              rows_b, gsem)
        for g in range(CH // 16):
            s = g * 16
            isrc = src_v[pl.ds(s, 16)]
            idst = dst2_v[g // (SUB // 16), pl.ds((s % SUB), 16)]
            sj = plsc.load_gather(sj_v, [isrc])
            si = plsc.load_gather(si_v, [idst])
            p2_v[g // (SUB // 16), pl.ds(s % SUB, 16)] = (
                jnp.exp(si + sj + se_v[pl.ds(s, 16)]))
        ph = [pltpu.async_copy(p2_v.at[j], ssum_sh.at[dst2_v.at[j]], psem,
                               add=True)
              for j in range(NSUB)]
        gh = [g0, g1, None, None, None]
        sh = [None] * NSUB
        for j in range(NSUB):
            b = j % 2
            if j >= 2:
                sh[j - 2].wait()
                gh[j] = pltpu.async_copy(
                    xl_hbm.at[src_v.at[pl.ds(j * SUB, SUB)]], bufs[b],
                    gsem)
            gh[j].wait()
            _scale_sub(bufs[b], p2_v, j)
            sh[j] = pltpu.async_copy(bufs[b], acc_sh.at[dst2_v.at[j]],
                                     sems[b], add=True)
        sh[NSUB - 2].wait()
        sh[NSUB - 1].wait()
        for h in ph:
            h.wait()
        return carry

    lax.fori_loop(0, NCH, chunk, 0)

    plsc.subcore_barrier()

    def drain_blk(j, carry):
        off = sid * RPT + j * BCH
        pltpu.sync_copy(acc_sh.at[pl.ds(off, BCH)], rows_a.at[pl.ds(0, BCH)])
        pltpu.sync_copy(rows_a.at[pl.ds(0, BCH)],
                        acc_hbm.at[cid, pl.ds(off, BCH)])
        return carry

    lax.fori_loop(0, NBCH, drain_blk, 0)

    @pl.when(sid == 15)
    def _():
        off = 15 * RPT + NBCH * BCH
        pltpu.sync_copy(acc_sh.at[pl.ds(off, 16)], rows_a.at[pl.ds(0, 16)])
        pltpu.sync_copy(rows_a.at[pl.ds(0, 16)],
                        acc_hbm.at[cid, pl.ds(off, 16)])

    @pl.when(sid < 10)
    def _():
        pltpu.sync_copy(ssum_sh.at[pl.ds(sid * SPT, SPT)], ssb_v)

    @pl.when(jnp.logical_and(sid < 10, cid == 0))
    def _():
        pltpu.sync_copy(ssb_v, s0_hbm.at[pl.ds(sid * SPT, SPT)])

    @pl.when(jnp.logical_and(sid < 10, cid == 1))
    def _():
        pltpu.sync_copy(ssb_v, s1_hbm.at[pl.ds(sid * SPT, SPT)])


@functools.cache
def _sc_call():
    return pl.kernel(
        _sc_body,
        mesh=plsc.VectorSubcoreMesh(core_axis_name="c", subcore_axis_name="s"),
        compiler_params=pltpu.CompilerParams(needs_layout_passes=False),
        out_type=[jax.ShapeDtypeStruct((NC, N, C), jnp.float32),
                  jax.ShapeDtypeStruct((N,), jnp.float32),
                  jax.ShapeDtypeStruct((N,), jnp.float32)],
        scratch_types=[
            pltpu.VMEM((N,), jnp.float32),
            pltpu.VMEM((N,), jnp.float32),
            pltpu.VMEM((CH,), jnp.int32),
            pltpu.VMEM((NSUB, SUB), jnp.int32),
            pltpu.VMEM((CH,), jnp.float32),
            pltpu.VMEM((NSUB, SUB), jnp.float32),
            pltpu.VMEM((SUB, C), jnp.float32),
            pltpu.VMEM((SUB, C), jnp.float32),
            pltpu.VMEM((SPT,), jnp.float32),
            pltpu.VMEM_SHARED((N, C), jnp.float32),
            pltpu.VMEM_SHARED((N,), jnp.float32),
            pltpu.SemaphoreType.DMA,
            pltpu.SemaphoreType.DMA,
            pltpu.SemaphoreType.DMA,
            pltpu.SemaphoreType.DMA,
        ],
    )


def kernel(x, edge_attr, edge_index, W_l, b_l, W_e, b_e, att, W_no, b_no,
           W_eo, b_eo):
    ea3 = edge_attr.reshape(EP, 8, DE)
    eye8 = jnp.eye(8, dtype=jnp.float32)
    wbd = jnp.kron(eye8, W_e.T)
    bt = jnp.tile(b_e, 8)
    att_flat = att.reshape(ATT)
    att_t = jnp.tile(att_flat[2 * C:], 8)
    sel = jnp.kron(eye8, jnp.ones((DE, 1), jnp.float32))
    wbdo = jnp.kron(eye8, W_eo.T)
    bot = jnp.tile(b_eo, 8)

    xl, si, sj = _node_call(x, W_l, b_l, att)
    se8 = _se_call(ea3, wbd, bt, att_t, sel)
    se = se8.reshape(E)
    src = edge_index[0]
    dst3 = edge_index[1].reshape(E // CH, NSUB, SUB)
    zr = jnp.zeros((BCH, C), jnp.float32)
    zs = jnp.zeros((SPT,), jnp.float32)
    acc2, ssum0, ssum1 = _sc_call()(xl, si, sj, se, src, dst3, zr, zs)
    eo3 = _eo_call(ea3, wbd, bt, wbdo, bot)
    e_out = eo3.reshape(E, EO)
    out = _out_call(acc2, ssum0, ssum1, W_no, b_no)
    return (out, e_out)


# speedup vs baseline: 1.1919x; 1.0314x over previous
import functools

import jax
import jax.numpy as jnp
from jax import lax
from jax.experimental import pallas as pl
from jax.experimental.pallas import tpu as pltpu
from jax.experimental.pallas import tpu_sc as plsc

N = 10000
E = 320000
DF = 128
C = 128
DE = 16
EO = 16
ATT = 2 * C + EO
NEG = 0.2

NC = 2
NS = 16
NW = NC * NS
EPT = E // NW
SUB = 80
NSUB = 5
CH = SUB * NSUB
NCH = EPT // CH
RPT = 624
RPT_LAST = N - 15 * RPT
BCH = 48
NBCH = RPT // BCH
SPT = N // 10

EB = 8000
_PREC = lax.Precision.DEFAULT
_EPREC = lax.Precision.DEFAULT


def _leaky(v):
    return jnp.where(v >= 0, v, NEG * v)


def _node_body(x_ref, wl_ref, bl_ref, att_ref, xl_ref, si_ref, sj_ref):
    xl = lax.dot_general(x_ref[...], wl_ref[...], (((1,), (1,)), ((), ())),
                         precision=_PREC) + bl_ref[...][None, :]
    xl_ref[...] = xl
    lr = _leaky(xl)
    a = att_ref[...].reshape(ATT)
    si_ref[...] = lr @ a[:C]
    sj_ref[...] = lr @ a[C:2 * C]


def _se_body(ea_ref, wbd_ref, bt_ref, att_ref, sel_ref, se_ref):
    ea = ea_ref[...].reshape(EBS, 8 * DE)
    e = lax.dot_general(ea, wbd_ref[...], (((1,), (0,)), ((), ())),
                        precision=_EPREC) + bt_ref[...][None, :]
    se_ref[...] = lax.dot_general(_leaky(e) * att_ref[...][None, :],
                                  sel_ref[...], (((1,), (0,)), ((), ())),
                                  precision=_EPREC)


def _eo_body(ea_ref, wbd_ref, bt_ref, wbdo_ref, bot_ref, eo_ref):
    ea = ea_ref[...].reshape(EBO, 8 * DE)
    e = lax.dot_general(ea, wbd_ref[...], (((1,), (0,)), ((), ())),
                        precision=_EPREC) + bt_ref[...][None, :]
    eo = lax.dot_general(e, wbdo_ref[...], (((1,), (0,)), ((), ())),
                         precision=_EPREC) + bot_ref[...][None, :]
    eo_ref[...] = eo.reshape(EBO, 8, EO)


def _out_body(acc_ref, s0_ref, s1_ref, wno_ref, bno_ref, out_ref):
    acc = acc_ref[0] + acc_ref[1]
    ssum = s0_ref[...] + s1_ref[...]
    scale = 1.0 / (ssum + 1e-16)
    out_ref[...] = lax.dot_general(acc * scale[:, None], wno_ref[...],
                                   (((1,), (1,)), ((), ())),
                                   precision=_PREC) + bno_ref[...][None, :]


_node_call = pl.pallas_call(
    _node_body,
    out_shape=[jax.ShapeDtypeStruct((N, C), jnp.float32),
               jax.ShapeDtypeStruct((N,), jnp.float32),
               jax.ShapeDtypeStruct((N,), jnp.float32)],
)

EP = E // 8
EBS = 1000

_se_call = pl.pallas_call(
    _se_body,
    grid=(EP // EBS,),
    in_specs=[pl.BlockSpec((EBS, 8, DE), lambda i: (i, 0, 0)),
              pl.BlockSpec((128, 128), lambda i: (0, 0)),
              pl.BlockSpec((128,), lambda i: (0,)),
              pl.BlockSpec((128,), lambda i: (0,)),
              pl.BlockSpec((128, 8), lambda i: (0, 0))],
    out_specs=pl.BlockSpec((EBS, 8), lambda i: (i, 0)),
    out_shape=jax.ShapeDtypeStruct((EP, 8), jnp.float32),
)

EBO = 1000

_eo_call = pl.pallas_call(
    _eo_body,
    grid=(EP // EBO,),
    in_specs=[pl.BlockSpec((EBO, 8, DE), lambda i: (i, 0, 0)),
              pl.BlockSpec((128, 128), lambda i: (0, 0)),
              pl.BlockSpec((128,), lambda i: (0,)),
              pl.BlockSpec((128, 128), lambda i: (0, 0)),
              pl.BlockSpec((128,), lambda i: (0,))],
    out_specs=pl.BlockSpec((EBO, 8, EO), lambda i: (i, 0, 0)),
    out_shape=jax.ShapeDtypeStruct((EP, 8, EO), jnp.float32),
)

_out_call = pl.pallas_call(
    _out_body,
    out_shape=jax.ShapeDtypeStruct((N, C), jnp.float32),
)


def _scale_sub(rows_ref, p2_ref, j):
    def scale_grp(g, c2):
        p16 = p2_ref[j, pl.ds(g * 16, 16)]
        for l in range(16):
            b = g * 16 + l
            pb = p16[l]
            for cc in range(C // 16):
                rows_ref[b, pl.ds(cc * 16, 16)] = (
                    rows_ref[b, pl.ds(cc * 16, 16)] * pb)
        return c2

    lax.fori_loop(0, SUB // 16, scale_grp, 0)


def _sc_body(xl_hbm, si_hbm, sj_hbm, se_hbm, src_hbm, dst3_hbm, zr_hbm,
             zs_hbm, acc_hbm, s0_hbm, s1_hbm,
             si_v, sj_v, src_v, dst2_v, se_v, p2_v, rows_a, rows_b, ssb_v,
             acc_sh, ssum_sh, gsem, sem_a, sem_b, psem):
    cid = lax.axis_index("c")
    sid = lax.axis_index("s")
    wid = cid * NS + sid

    pltpu.sync_copy(si_hbm, si_v)
    pltpu.sync_copy(sj_hbm, sj_v)

    pltpu.sync_copy(zr_hbm, rows_a.at[pl.ds(0, BCH)])

    def zero_blk(j, carry):
        pltpu.sync_copy(rows_a.at[pl.ds(0, BCH)],
                        acc_sh.at[pl.ds(sid * RPT + j * BCH, BCH)])
        return carry

    lax.fori_loop(0, NBCH, zero_blk, 0)

    @pl.when(sid == 15)
    def _():
        pltpu.sync_copy(rows_a.at[pl.ds(0, 16)],
                        acc_sh.at[pl.ds(15 * RPT + NBCH * BCH, 16)])

    @pl.when(sid < 10)
    def _():
        pltpu.sync_copy(zs_hbm, ssb_v)
        pltpu.sync_copy(ssb_v, ssum_sh.at[pl.ds(sid * SPT, SPT)])

    plsc.subcore_barrier()

    bufs = (rows_a, rows_b)
    sems = (sem_a, sem_b)

    def chunk(ch, carry):
        base = wid * EPT + ch * CH
        l0 = pltpu.async_copy(src_hbm.at[pl.ds(base, CH)], src_v, gsem)
        l1 = pltpu.async_copy(dst3_hbm.at[wid * NCH + ch], dst2_v, gsem)
        l2 = pltpu.async_copy(se_hbm.at[pl.ds(base, CH)], se_v, gsem)
        l0.wait()
        l1.wait()
        l2.wait()
        g0 = pltpu.async_copy(xl_hbm.at[src_v.at[pl.ds(0, SUB)]],
                              rows_a, gsem)
        g1 = pltpu.async_copy(xl_hbm.at[src_v.at[pl.ds(SUB, SUB)]],
                              rows_b, gsem)
        for g in range(CH // 16):
            s = g * 16
            isrc = src_v[pl.ds(s, 16)]
            idst = dst2_v[g // (SUB // 16), pl.ds((s % SUB), 16)]
            sj = plsc.load_gather(sj_v, [isrc])
            si = plsc.load_gather(si_v, [idst])
            p2_v[g // (SUB // 16), pl.ds(s % SUB, 16)] = (
                jnp.exp(si + sj + se_v[pl.ds(s, 16)]))
        ph = [pltpu.async_copy(p2_v.at[j], ssum_sh.at[dst2_v.at[j]], psem,
                               add=True)
              for j in range(NSUB)]
        gh = [g0, g1, None, None, None]
        sh = [None] * NSUB
        for j in range(NSUB):
            b = j % 2
            if j >= 2:
                sh[j - 2].wait()
                gh[j] = pltpu.async_copy(
                    xl_hbm.at[src_v.at[pl.ds(j * SUB, SUB)]], bufs[b],
                    gsem)
            gh[j].wait()
            _scale_sub(bufs[b], p2_v, j)
            sh[j] = pltpu.async_copy(bufs[b], acc_sh.at[dst2_v.at[j]],
                                     sems[b], add=True)
        sh[NSUB - 2].wait()
        sh[NSUB - 1].wait()
        for h in ph:
            h.wait()
        return carry

    lax.fori_loop(0, NCH, chunk, 0)

    plsc.subcore_barrier()

    def drain_blk(j, carry):
        off = sid * RPT + j * BCH
        pltpu.sync_copy(acc_sh.at[pl.ds(off, BCH)], rows_a.at[pl.ds(0, BCH)])
        pltpu.sync_copy(rows_a.at[pl.ds(0, BCH)],
                        acc_hbm.at[cid, pl.ds(off, BCH)])
        return carry

    lax.fori_loop(0, NBCH, drain_blk, 0)

    @pl.when(sid == 15)
    def _():
        off = 15 * RPT + NBCH * BCH
        pltpu.sync_copy(acc_sh.at[pl.ds(off, 16)], rows_a.at[pl.ds(0, 16)])
        pltpu.sync_copy(rows_a.at[pl.ds(0, 16)],
                        acc_hbm.at[cid, pl.ds(off, 16)])

    @pl.when(sid < 10)
    def _():
        pltpu.sync_copy(ssum_sh.at[pl.ds(sid * SPT, SPT)], ssb_v)

    @pl.when(jnp.logical_and(sid < 10, cid == 0))
    def _():
        pltpu.sync_copy(ssb_v, s0_hbm.at[pl.ds(sid * SPT, SPT)])

    @pl.when(jnp.logical_and(sid < 10, cid == 1))
    def _():
        pltpu.sync_copy(ssb_v, s1_hbm.at[pl.ds(sid * SPT, SPT)])


@functools.cache
def _sc_call():
    return pl.kernel(
        _sc_body,
        mesh=plsc.VectorSubcoreMesh(core_axis_name="c", subcore_axis_name="s"),
        compiler_params=pltpu.CompilerParams(needs_layout_passes=False),
        out_type=[jax.ShapeDtypeStruct((NC, N, C), jnp.float32),
                  jax.ShapeDtypeStruct((N,), jnp.float32),
                  jax.ShapeDtypeStruct((N,), jnp.float32)],
        scratch_types=[
            pltpu.VMEM((N,), jnp.float32),
            pltpu.VMEM((N,), jnp.float32),
            pltpu.VMEM((CH,), jnp.int32),
            pltpu.VMEM((NSUB, SUB), jnp.int32),
            pltpu.VMEM((CH,), jnp.float32),
            pltpu.VMEM((NSUB, SUB), jnp.float32),
            pltpu.VMEM((SUB, C), jnp.float32),
            pltpu.VMEM((SUB, C), jnp.float32),
            pltpu.VMEM((SPT,), jnp.float32),
            pltpu.VMEM_SHARED((N, C), jnp.float32),
            pltpu.VMEM_SHARED((N,), jnp.float32),
            pltpu.SemaphoreType.DMA,
            pltpu.SemaphoreType.DMA,
            pltpu.SemaphoreType.DMA,
            pltpu.SemaphoreType.DMA,
        ],
    )


def kernel(x, edge_attr, edge_index, W_l, b_l, W_e, b_e, att, W_no, b_no,
           W_eo, b_eo):
    ea3 = edge_attr.reshape(EP, 8, DE)
    eye8 = jnp.eye(8, dtype=jnp.float32)
    wbd = jnp.kron(eye8, W_e.T)
    bt = jnp.tile(b_e, 8)
    att_flat = att.reshape(ATT)
    att_t = jnp.tile(att_flat[2 * C:], 8)
    sel = jnp.kron(eye8, jnp.ones((DE, 1), jnp.float32))
    wbdo = jnp.kron(eye8, W_eo.T)
    bot = jnp.tile(b_eo, 8)

    xl, si, sj = _node_call(x, W_l, b_l, att)
    se8 = _se_call(ea3, wbd, bt, att_t, sel)
    se = se8.reshape(E)
    src = edge_index[0]
    dst3 = edge_index[1].reshape(E // CH, NSUB, SUB)
    zr = jnp.zeros((BCH, C), jnp.float32)
    zs = jnp.zeros((SPT,), jnp.float32)
    acc2, ssum0, ssum1 = _sc_call()(xl, si, sj, se, src, dst3, zr, zs)
    eo3 = _eo_call(ea3, wbd, bt, wbdo, bot)
    e_out = eo3.reshape(E, EO)
    out = _out_call(acc2, ssum0, ssum1, W_no, b_no)
    return (out, e_out)
